# baseline (speedup 1.0000x reference)
import functools

import jax
import jax.numpy as jnp
from jax import lax
from jax.experimental import pallas as pl
from jax.experimental.pallas import tpu as pltpu
from jax.experimental.pallas import tpu_sc as plsc

_N = 10000
_E = 160000
_F = 256
_H = 256
_G = 64
_EPS = 1e-5

_NSUB = 16
_HALF = _H // 2
_EPT = _E // _NSUB
_K = 112
_NFULL = _EPT // _K
_TAIL = _EPT - _NFULL * _K
_ZC = 80
_RPT = 640
_RPT_LAST = _N - 15 * _RPT


def _segsum_body(xlo, xhi, src3, dst3, srct, dstt, out_lo, out_hi,
                 sidx, didx, stail, dtail, rows0, rows1, shared, sem0, sem1):
    c = lax.axis_index("c")
    s = lax.axis_index("s")

    pltpu.async_copy(src3.at[s], sidx, sem0)
    pltpu.async_copy(dst3.at[s], didx, sem1)
    pltpu.sync_copy(srct.at[s], stail.at[0])
    pltpu.sync_copy(dstt.at[s], dtail.at[0])

    zvec = jnp.zeros((16,), jnp.float32)

    def zrow(r, carry):
        for k in range(_HALF // 16):
            rows0[r, pl.ds(k * 16, 16)] = zvec
        return carry
    lax.fori_loop(0, _ZC, zrow, 0)
    ncopy = jnp.where(s == _NSUB - 1, _RPT_LAST // _ZC, _RPT // _ZC)

    def zcopy(t, carry):
        pltpu.sync_copy(rows0.at[pl.ds(0, _ZC)],
                        shared.at[pl.ds(s * _RPT + t * _ZC, _ZC)])
        return carry
    lax.fori_loop(0, ncopy, zcopy, 0)
    plsc.subcore_barrier()

    pltpu.make_async_copy(src3.at[s], sidx, sem0).wait()
    pltpu.make_async_copy(dst3.at[s], didx, sem1).wait()

    def half(tab, outh):
        def gather(j, buf, sem):
            pltpu.async_copy(tab.at[sidx.at[j]], buf, sem)

        def wait(j, buf, sem):
            pltpu.make_async_copy(tab.at[sidx.at[j]], buf, sem).wait()

        def scatter(j, buf):
            pltpu.sync_copy(buf, shared.at[didx.at[j]], add=True)

        gather(0, rows0, sem0)

        def chunk2(i, carry):
            j = 2 * i
            gather(j + 1, rows1, sem1)
            wait(j, rows0, sem0)
            scatter(j, rows0)
            gather(j + 2, rows0, sem0)
            wait(j + 1, rows1, sem1)
            scatter(j + 1, rows1)
            return carry
        lax.fori_loop(0, (_NFULL - 1) // 2, chunk2, 0)
        pltpu.async_copy(tab.at[stail.at[0]], rows1.at[pl.ds(0, _TAIL)], sem1)
        wait(_NFULL - 1, rows0, sem0)
        scatter(_NFULL - 1, rows0)
        pltpu.make_async_copy(tab.at[stail.at[0]], rows1.at[pl.ds(0, _TAIL)],
                              sem1).wait()
        pltpu.sync_copy(rows1.at[pl.ds(0, _TAIL)], shared.at[dtail.at[0]],
                        add=True)
        plsc.subcore_barrier()

        @pl.when(s < _NSUB - 1)
        def _():
            pltpu.sync_copy(shared.at[pl.ds(s * _RPT, _RPT)],
                            outh.at[pl.ds(s * _RPT, _RPT)])

        @pl.when(s == _NSUB - 1)
        def _():
            pltpu.sync_copy(shared.at[pl.ds(15 * _RPT, _RPT_LAST)],
                            outh.at[pl.ds(15 * _RPT, _RPT_LAST)])

    @pl.when(c == 0)
    def _():
        half(xlo, out_lo)

    @pl.when(c == 1)
    def _():
        half(xhi, out_hi)


@functools.partial(jax.jit, static_argnums=())
def _segsum(x_lo, x_hi, src3, dst3, srct, dstt):
    mesh = plsc.VectorSubcoreMesh(core_axis_name="c", subcore_axis_name="s")
    f = pl.kernel(
        _segsum_body,
        out_type=[
            jax.ShapeDtypeStruct((_N, _HALF), jnp.float32),
            jax.ShapeDtypeStruct((_N, _HALF), jnp.float32),
        ],
        scratch_types=[
            pltpu.VMEM((_NFULL, _K), jnp.int32),
            pltpu.VMEM((_NFULL, _K), jnp.int32),
            pltpu.VMEM((1, _TAIL), jnp.int32),
            pltpu.VMEM((1, _TAIL), jnp.int32),
            pltpu.VMEM((_K, _HALF), jnp.float32),
            pltpu.VMEM((_K, _HALF), jnp.float32),
            pltpu.VMEM_SHARED((_N, _HALF), jnp.float32),
            pltpu.SemaphoreType.DMA,
            pltpu.SemaphoreType.DMA,
        ],
        mesh=mesh,
        compiler_params=pltpu.CompilerParams(use_tc_tiling_on_sc=False),
    )
    return f(x_lo, x_hi, src3, dst3, srct, dstt)


def _premul_tc(x_ref, wa_ref, olo_ref, ohi_ref):
    y = jnp.dot(x_ref[...].astype(jnp.bfloat16), wa_ref[...].astype(jnp.bfloat16),
                preferred_element_type=jnp.float32)
    olo_ref[...] = y[:, :_HALF]
    ohi_ref[...] = y[:, _HALF:]


def _layer1_tc(ylo_ref, yhi_ref, alo_ref, ahi_ref, ba_ref, wb_ref, bb_ref,
               g_ref, be_ref, w2a_ref, olo_ref, ohi_ref):
    zlo = jnp.maximum(ylo_ref[...] + alo_ref[...] + ba_ref[:, :_HALF], 0.0)
    zhi = jnp.maximum(yhi_ref[...] + ahi_ref[...] + ba_ref[:, _HALF:], 0.0)
    wb = wb_ref[...].astype(jnp.bfloat16)
    z = (jnp.dot(zlo.astype(jnp.bfloat16), wb[:_HALF, :],
                 preferred_element_type=jnp.float32)
         + jnp.dot(zhi.astype(jnp.bfloat16), wb[_HALF:, :],
                   preferred_element_type=jnp.float32) + bb_ref[...])
    mean = jnp.mean(z, axis=0, keepdims=True)
    var = jnp.mean(z * z, axis=0, keepdims=True) - mean * mean
    h = (z - mean) * lax.rsqrt(var + _EPS) * g_ref[...] + be_ref[...]
    h = jnp.maximum(h, 0.0)
    y2 = jnp.dot(h.astype(jnp.bfloat16), w2a_ref[...].astype(jnp.bfloat16),
                 preferred_element_type=jnp.float32)
    olo_ref[...] = y2[:, :_HALF]
    ohi_ref[...] = y2[:, _HALF:]


def _layer2_tc(ylo_ref, yhi_ref, alo_ref, ahi_ref, ba_ref, wb_ref, bb_ref,
               g_ref, be_ref, batch_ref, out_ref):
    zlo = jnp.maximum(ylo_ref[...] + alo_ref[...] + ba_ref[:, :_HALF], 0.0)
    zhi = jnp.maximum(yhi_ref[...] + ahi_ref[...] + ba_ref[:, _HALF:], 0.0)
    wb = wb_ref[...].astype(jnp.bfloat16)
    z = (jnp.dot(zlo.astype(jnp.bfloat16), wb[:_HALF, :],
                 preferred_element_type=jnp.float32)
         + jnp.dot(zhi.astype(jnp.bfloat16), wb[_HALF:, :],
                   preferred_element_type=jnp.float32) + bb_ref[...])
    mean = jnp.mean(z, axis=0, keepdims=True)
    var = jnp.mean(z * z, axis=0, keepdims=True) - mean * mean
    h = (z - mean) * lax.rsqrt(var + _EPS) * g_ref[...] + be_ref[...]
    h = jnp.maximum(h, 0.0)
    gids = lax.broadcasted_iota(jnp.int32, (_G, 1), 0)
    mask = (batch_ref[...] == gids).astype(jnp.float32)
    cnt = jnp.maximum(jnp.sum(mask, axis=1, keepdims=True), 1.0)
    out_ref[...] = jnp.dot(mask.astype(jnp.bfloat16), h.astype(jnp.bfloat16),
                           preferred_element_type=jnp.float32) / cnt


def kernel(x, edge_index, batch, W1a, b1a, W1b, b1b, g1, beta1,
           W2a, b2a, W2b, b2b, g2, beta2):
    e_src = edge_index[0].reshape(_NSUB, _EPT)
    e_dst = edge_index[1].reshape(_NSUB, _EPT)
    src3 = e_src[:, :_NFULL * _K].reshape(_NSUB, _NFULL, _K)
    dst3 = e_dst[:, :_NFULL * _K].reshape(_NSUB, _NFULL, _K)
    srct = e_src[:, _NFULL * _K:]
    dstt = e_dst[:, _NFULL * _K:]
    batch2 = batch.reshape(1, _N)

    two_half = [
        jax.ShapeDtypeStruct((_N, _HALF), jnp.float32),
        jax.ShapeDtypeStruct((_N, _HALF), jnp.float32),
    ]

    y1lo, y1hi = pl.pallas_call(_premul_tc, out_shape=two_half)(x, W1a)

    a1lo, a1hi = _segsum(y1lo, y1hi, src3, dst3, srct, dstt)

    y2lo, y2hi = pl.pallas_call(_layer1_tc, out_shape=two_half)(
        y1lo, y1hi, a1lo, a1hi, b1a.reshape(1, _H), W1b, b1b.reshape(1, _H),
        g1.reshape(1, _H), beta1.reshape(1, _H), W2a)

    a2lo, a2hi = _segsum(y2lo, y2hi, src3, dst3, srct, dstt)

    out = pl.pallas_call(
        _layer2_tc,
        out_shape=jax.ShapeDtypeStruct((_G, _H), jnp.float32),
    )(y2lo, y2hi, a2lo, a2hi, b2a.reshape(1, _H), W2b, b2b.reshape(1, _H),
      g2.reshape(1, _H), beta2.reshape(1, _H), batch2)

    return out


# speedup vs baseline: 1.0030x; 1.0030x over previous
import functools

import jax
import jax.numpy as jnp
from jax import lax
from jax.experimental import pallas as pl
from jax.experimental.pallas import tpu as pltpu
from jax.experimental.pallas import tpu_sc as plsc

_N = 10000
_E = 160000
_F = 256
_H = 256
_G = 64
_EPS = 1e-5

_NSUB = 16
_HALF = _H // 2
_EPT = _E // _NSUB
_K = 112
_NFULL = _EPT // _K
_TAIL = _EPT - _NFULL * _K
_ZC = 80
_RPT = 640
_RPT_LAST = _N - 15 * _RPT


def _segsum_body(xlo, xhi, src3, dst3, srct, dstt, out_lo, out_hi,
                 sidx, didx, stail, dtail, rows0, rows1, shared,
                 sem0, sem1, zsem):
    c = lax.axis_index("c")
    s = lax.axis_index("s")

    pltpu.async_copy(src3.at[s], sidx, sem0)
    pltpu.async_copy(dst3.at[s], didx, sem1)
    pltpu.sync_copy(srct.at[s], stail.at[0])
    pltpu.sync_copy(dstt.at[s], dtail.at[0])

    zvec = jnp.zeros((16,), jnp.float32)

    def zrow(r, carry):
        for k in range(_HALF // 16):
            rows0[r, pl.ds(k * 16, 16)] = zvec
        return carry
    lax.fori_loop(0, _ZC, zrow, 0)
    ncopy = jnp.where(s == _NSUB - 1, _RPT_LAST // _ZC, _RPT // _ZC)

    def zcopy(t, carry):
        pltpu.async_copy(rows0.at[pl.ds(0, _ZC)],
                         shared.at[pl.ds(s * _RPT + t * _ZC, _ZC)], zsem)
        return carry
    lax.fori_loop(0, ncopy, zcopy, 0)

    pltpu.make_async_copy(src3.at[s], sidx, sem0).wait()
    pltpu.make_async_copy(dst3.at[s], didx, sem1).wait()

    def zwait(t, carry):
        pltpu.make_async_copy(rows0.at[pl.ds(0, _ZC)],
                              shared.at[pl.ds(s * _RPT + t * _ZC, _ZC)],
                              zsem).wait()
        return carry
    lax.fori_loop(0, ncopy, zwait, 0)
    plsc.subcore_barrier()

    def half(tab, outh):
        def gather(j, buf, sem):
            pltpu.async_copy(tab.at[sidx.at[j]], buf, sem)

        def wait(j, buf, sem):
            pltpu.make_async_copy(tab.at[sidx.at[j]], buf, sem).wait()

        def scatter(j, buf):
            pltpu.sync_copy(buf, shared.at[didx.at[j]], add=True)

        gather(0, rows0, sem0)

        def chunk2(i, carry):
            j = 2 * i
            gather(j + 1, rows1, sem1)
            wait(j, rows0, sem0)
            scatter(j, rows0)
            gather(j + 2, rows0, sem0)
            wait(j + 1, rows1, sem1)
            scatter(j + 1, rows1)
            return carry
        lax.fori_loop(0, (_NFULL - 1) // 2, chunk2, 0)
        pltpu.async_copy(tab.at[stail.at[0]], rows1.at[pl.ds(0, _TAIL)], sem1)
        wait(_NFULL - 1, rows0, sem0)
        scatter(_NFULL - 1, rows0)
        pltpu.make_async_copy(tab.at[stail.at[0]], rows1.at[pl.ds(0, _TAIL)],
                              sem1).wait()
        pltpu.sync_copy(rows1.at[pl.ds(0, _TAIL)], shared.at[dtail.at[0]],
                        add=True)
        plsc.subcore_barrier()

        @pl.when(s < _NSUB - 1)
        def _():
            pltpu.sync_copy(shared.at[pl.ds(s * _RPT, _RPT)],
                            outh.at[pl.ds(s * _RPT, _RPT)])

        @pl.when(s == _NSUB - 1)
        def _():
            pltpu.sync_copy(shared.at[pl.ds(15 * _RPT, _RPT_LAST)],
                            outh.at[pl.ds(15 * _RPT, _RPT_LAST)])

    @pl.when(c == 0)
    def _():
        half(xlo, out_lo)

    @pl.when(c == 1)
    def _():
        half(xhi, out_hi)


@functools.partial(jax.jit, static_argnums=())
def _segsum(x_lo, x_hi, src3, dst3, srct, dstt):
    mesh = plsc.VectorSubcoreMesh(core_axis_name="c", subcore_axis_name="s")
    f = pl.kernel(
        _segsum_body,
        out_type=[
            jax.ShapeDtypeStruct((_N, _HALF), jnp.float32),
            jax.ShapeDtypeStruct((_N, _HALF), jnp.float32),
        ],
        scratch_types=[
            pltpu.VMEM((_NFULL, _K), jnp.int32),
            pltpu.VMEM((_NFULL, _K), jnp.int32),
            pltpu.VMEM((1, _TAIL), jnp.int32),
            pltpu.VMEM((1, _TAIL), jnp.int32),
            pltpu.VMEM((_K, _HALF), jnp.float32),
            pltpu.VMEM((_K, _HALF), jnp.float32),
            pltpu.VMEM_SHARED((_N, _HALF), jnp.float32),
            pltpu.SemaphoreType.DMA,
            pltpu.SemaphoreType.DMA,
            pltpu.SemaphoreType.DMA,
        ],
        mesh=mesh,
        compiler_params=pltpu.CompilerParams(use_tc_tiling_on_sc=False),
    )
    return f(x_lo, x_hi, src3, dst3, srct, dstt)


def _premul_tc(x_ref, wa_ref, olo_ref, ohi_ref):
    y = jnp.dot(x_ref[...].astype(jnp.bfloat16), wa_ref[...].astype(jnp.bfloat16),
                preferred_element_type=jnp.float32)
    olo_ref[...] = y[:, :_HALF]
    ohi_ref[...] = y[:, _HALF:]


def _layer1_tc(ylo_ref, yhi_ref, alo_ref, ahi_ref, ba_ref, wb_ref, bb_ref,
               g_ref, be_ref, w2a_ref, olo_ref, ohi_ref):
    zlo = jnp.maximum(ylo_ref[...] + alo_ref[...] + ba_ref[:, :_HALF], 0.0)
    zhi = jnp.maximum(yhi_ref[...] + ahi_ref[...] + ba_ref[:, _HALF:], 0.0)
    wb = wb_ref[...].astype(jnp.bfloat16)
    z = (jnp.dot(zlo.astype(jnp.bfloat16), wb[:_HALF, :],
                 preferred_element_type=jnp.float32)
         + jnp.dot(zhi.astype(jnp.bfloat16), wb[_HALF:, :],
                   preferred_element_type=jnp.float32) + bb_ref[...])
    mean = jnp.mean(z, axis=0, keepdims=True)
    var = jnp.mean(z * z, axis=0, keepdims=True) - mean * mean
    h = (z - mean) * lax.rsqrt(var + _EPS) * g_ref[...] + be_ref[...]
    h = jnp.maximum(h, 0.0)
    y2 = jnp.dot(h.astype(jnp.bfloat16), w2a_ref[...].astype(jnp.bfloat16),
                 preferred_element_type=jnp.float32)
    olo_ref[...] = y2[:, :_HALF]
    ohi_ref[...] = y2[:, _HALF:]


def _layer2_tc(ylo_ref, yhi_ref, alo_ref, ahi_ref, ba_ref, wb_ref, bb_ref,
               g_ref, be_ref, batch_ref, out_ref):
    zlo = jnp.maximum(ylo_ref[...] + alo_ref[...] + ba_ref[:, :_HALF], 0.0)
    zhi = jnp.maximum(yhi_ref[...] + ahi_ref[...] + ba_ref[:, _HALF:], 0.0)
    wb = wb_ref[...].astype(jnp.bfloat16)
    z = (jnp.dot(zlo.astype(jnp.bfloat16), wb[:_HALF, :],
                 preferred_element_type=jnp.float32)
         + jnp.dot(zhi.astype(jnp.bfloat16), wb[_HALF:, :],
                   preferred_element_type=jnp.float32) + bb_ref[...])
    mean = jnp.mean(z, axis=0, keepdims=True)
    var = jnp.mean(z * z, axis=0, keepdims=True) - mean * mean
    h = (z - mean) * lax.rsqrt(var + _EPS) * g_ref[...] + be_ref[...]
    h = jnp.maximum(h, 0.0)
    gids = lax.broadcasted_iota(jnp.int32, (_G, 1), 0)
    mask = (batch_ref[...] == gids).astype(jnp.float32)
    cnt = jnp.maximum(jnp.sum(mask, axis=1, keepdims=True), 1.0)
    out_ref[...] = jnp.dot(mask.astype(jnp.bfloat16), h.astype(jnp.bfloat16),
                           preferred_element_type=jnp.float32) / cnt


def kernel(x, edge_index, batch, W1a, b1a, W1b, b1b, g1, beta1,
           W2a, b2a, W2b, b2b, g2, beta2):
    e_src = edge_index[0].reshape(_NSUB, _EPT)
    e_dst = edge_index[1].reshape(_NSUB, _EPT)
    src3 = e_src[:, :_NFULL * _K].reshape(_NSUB, _NFULL, _K)
    dst3 = e_dst[:, :_NFULL * _K].reshape(_NSUB, _NFULL, _K)
    srct = e_src[:, _NFULL * _K:]
    dstt = e_dst[:, _NFULL * _K:]
    batch2 = batch.reshape(1, _N)

    two_half = [
        jax.ShapeDtypeStruct((_N, _HALF), jnp.float32),
        jax.ShapeDtypeStruct((_N, _HALF), jnp.float32),
    ]

    y1lo, y1hi = pl.pallas_call(_premul_tc, out_shape=two_half)(x, W1a)

    a1lo, a1hi = _segsum(y1lo, y1hi, src3, dst3, srct, dstt)

    y2lo, y2hi = pl.pallas_call(_layer1_tc, out_shape=two_half)(
        y1lo, y1hi, a1lo, a1hi, b1a.reshape(1, _H), W1b, b1b.reshape(1, _H),
        g1.reshape(1, _H), beta1.reshape(1, _H), W2a)

    a2lo, a2hi = _segsum(y2lo, y2hi, src3, dst3, srct, dstt)

    out = pl.pallas_call(
        _layer2_tc,
        out_shape=jax.ShapeDtypeStruct((_G, _H), jnp.float32),
    )(y2lo, y2hi, a2lo, a2hi, b2a.reshape(1, _H), W2b, b2b.reshape(1, _H),
      g2.reshape(1, _H), beta2.reshape(1, _H), batch2)

    return out
